# baseline (speedup 1.0000x reference)
import functools

import jax
import jax.numpy as jnp
from jax import lax
from jax.experimental import pallas as pl
from jax.experimental.pallas import tpu as pltpu
from jax.experimental.pallas import tpu_sc as plsc

D = 64
B = 4096
S = 50
WIDE = 2 * D

NC = 2
NS = 16
NW = NC * NS

CB = 2
ROWS = CB * S
CHUNK_IDX = 112
B_PER_W = B // NW
CHUNKS_PER_W = B_PER_W // CB
IDX_PER_W = CHUNKS_PER_W * CHUNK_IDX


@functools.partial(
    pl.kernel,
    mesh=plsc.VectorSubcoreMesh(core_axis_name="c", subcore_axis_name="s"),
    out_type=jax.ShapeDtypeStruct((B, D), jnp.float32),
    compiler_params=pltpu.CompilerParams(needs_layout_passes=False),
    scratch_types=[
        pltpu.VMEM((IDX_PER_W,), jnp.int32),
        pltpu.VMEM((IDX_PER_W,), jnp.int32),
        pltpu.VMEM((ROWS, WIDE), jnp.float32),
        pltpu.VMEM((1, WIDE), jnp.float32),
        pltpu.VMEM((B_PER_W, D), jnp.float32),
        pltpu.SemaphoreType.DMA,
    ],
)
def _cbow_sc(idx_hbm, table_hbm, out_hbm, idx_v, idx2_v, rows_v, t0_v, out_v,
             sem):
    wid = lax.axis_index("s") * NC + lax.axis_index("c")

    pltpu.sync_copy(idx_hbm.at[pl.ds(wid * IDX_PER_W, IDX_PER_W)], idx_v)
    pltpu.sync_copy(table_hbm.at[pl.ds(0, 1)], t0_v)

    def halve_body(i, carry):
        o = pl.multiple_of(i * 16, 8)
        idx2_v[pl.ds(o, 16)] = lax.shift_right_logical(
            idx_v[pl.ds(o, 16)], 1
        )
        return carry

    lax.fori_loop(0, IDX_PER_W // 16, halve_body, None)

    t0 = [t0_v[0, pl.ds(q * 16, 16)] for q in range(4)]
    lane = lax.iota(jnp.int32, 16)
    inv_s = jnp.float32(1.0 / S)

    def popcnt(zb):
        return plsc.all_reduce_population_count(zb)

    def chunk_body(c, carry):
        off = pl.multiple_of(c * CHUNK_IDX, 8)
        idx_sl = idx2_v.at[pl.ds(off, ROWS)]
        pltpu.async_copy(table_hbm.at[idx_sl], rows_v, sem).wait()

        z = []
        ov = []
        for v in range(7):
            iv = idx_v[pl.ds(off + v * 16, 16)]
            z.append(iv == 0)
            ov.append(lax.shift_left(jnp.bitwise_and(iv, 1), 6))
        cnt0 = (
            popcnt(z[0])
            + popcnt(z[1])
            + popcnt(z[2])
            + popcnt(jnp.logical_and(z[3], lane < 2))
        ).astype(jnp.float32)
        cnt1 = (
            popcnt(jnp.logical_and(z[3], lane >= 2))
            + popcnt(z[4])
            + popcnt(z[5])
            + popcnt(jnp.logical_and(z[6], lane < 4))
        ).astype(jnp.float32)

        for b in range(CB):
            acc = [jnp.zeros((16,), jnp.float32) for _ in range(4)]
            for r in range(S):
                row_i = b * S + r
                o = ov[row_i // 16][row_i % 16]
                for q in range(4):
                    acc[q] = acc[q] + rows_v[row_i, pl.ds(o + q * 16, 16)]
            cnt = cnt0 if b == 0 else cnt1
            for q in range(4):
                out_v[c * CB + b, pl.ds(q * 16, 16)] = (
                    acc[q] - t0[q] * cnt
                ) * inv_s
        return carry

    lax.fori_loop(0, CHUNKS_PER_W, chunk_body, None)

    pltpu.sync_copy(out_v, out_hbm.at[pl.ds(wid * B_PER_W, B_PER_W)])


def kernel(indices, table):
    idx = indices.astype(jnp.int32).reshape(B // CB, ROWS)
    idx = jnp.pad(idx, ((0, 0), (0, CHUNK_IDX - ROWS)))
    return _cbow_sc(idx.reshape(-1), table.reshape(-1, WIDE))


# speedup vs baseline: 1.0935x; 1.0935x over previous
import functools

import jax
import jax.numpy as jnp
from jax import lax
from jax.experimental import pallas as pl
from jax.experimental.pallas import tpu as pltpu
from jax.experimental.pallas import tpu_sc as plsc

D = 64
B = 4096
S = 50

NC = 2
NS = 16
NW = NC * NS

CB = 2
ROWS = CB * S
CHUNK_IDX = 112
B_PER_W = B // NW
CHUNKS_PER_W = B_PER_W // CB
IDX_PER_W = CHUNKS_PER_W * CHUNK_IDX


@functools.partial(
    pl.kernel,
    mesh=plsc.VectorSubcoreMesh(core_axis_name="c", subcore_axis_name="s"),
    out_type=jax.ShapeDtypeStruct((B, D), jnp.float32),
    compiler_params=pltpu.CompilerParams(
        needs_layout_passes=False, use_tc_tiling_on_sc=False
    ),
    scratch_types=[
        pltpu.VMEM((IDX_PER_W,), jnp.int32),
        pltpu.VMEM((ROWS, D), jnp.float32),
        pltpu.VMEM((ROWS, D), jnp.float32),
        pltpu.VMEM((1, D), jnp.float32),
        pltpu.VMEM((B_PER_W, D), jnp.float32),
        pltpu.SemaphoreType.DMA,
        pltpu.SemaphoreType.DMA,
    ],
)
def _cbow_sc(idx_hbm, table_hbm, out_hbm, idx_v, rows_a, rows_b, t0_v, out_v,
             sem_a, sem_b):
    wid = lax.axis_index("s") * NC + lax.axis_index("c")

    pltpu.sync_copy(idx_hbm.at[pl.ds(wid * IDX_PER_W, IDX_PER_W)], idx_v)
    pltpu.sync_copy(table_hbm.at[pl.ds(0, 1)], t0_v)

    t0 = [t0_v[0, pl.ds(q * 16, 16)] for q in range(4)]
    lane = lax.iota(jnp.int32, 16)
    inv_s = jnp.float32(1.0 / S)

    def popcnt(zb):
        return plsc.all_reduce_population_count(zb)

    def start_gather(c, buf, sem):
        off = pl.multiple_of(c * CHUNK_IDX, 8)
        pltpu.async_copy(table_hbm.at[idx_v.at[pl.ds(off, ROWS)]], buf, sem)

    def wait_gather(buf, sem):
        pltpu.make_async_copy(
            table_hbm.at[idx_v.at[pl.ds(0, ROWS)]], buf, sem
        ).wait()

    def compute_chunk(c, rows_v):
        off = pl.multiple_of(c * CHUNK_IDX, 8)

        z = []
        for v in range(7):
            iv = idx_v[pl.ds(off + v * 16, 16)]
            z.append(iv == 0)
        cnt0 = (
            popcnt(z[0])
            + popcnt(z[1])
            + popcnt(z[2])
            + popcnt(jnp.logical_and(z[3], lane < 2))
        ).astype(jnp.float32)
        cnt1 = (
            popcnt(jnp.logical_and(z[3], lane >= 2))
            + popcnt(z[4])
            + popcnt(z[5])
            + popcnt(jnp.logical_and(z[6], lane < 4))
        ).astype(jnp.float32)

        for b in range(CB):
            acc = [jnp.zeros((16,), jnp.float32) for _ in range(4)]
            for r in range(S):
                row_i = b * S + r
                for q in range(4):
                    acc[q] = acc[q] + rows_v[row_i, pl.ds(q * 16, 16)]
            cnt = cnt0 if b == 0 else cnt1
            for q in range(4):
                out_v[c * CB + b, pl.ds(q * 16, 16)] = (
                    acc[q] - t0[q] * cnt
                ) * inv_s

    start_gather(0, rows_a, sem_a)

    def pair_body(i, carry):
        c0 = i * 2
        start_gather(c0 + 1, rows_b, sem_b)
        wait_gather(rows_a, sem_a)
        compute_chunk(c0, rows_a)

        @pl.when(c0 + 2 < CHUNKS_PER_W)
        def _():
            start_gather(c0 + 2, rows_a, sem_a)

        wait_gather(rows_b, sem_b)
        compute_chunk(c0 + 1, rows_b)
        return carry

    lax.fori_loop(0, CHUNKS_PER_W // 2, pair_body, None)

    pltpu.sync_copy(out_v, out_hbm.at[pl.ds(wid * B_PER_W, B_PER_W)])


def kernel(indices, table):
    idx = indices.astype(jnp.int32).reshape(B // CB, ROWS)
    idx = jnp.pad(idx, ((0, 0), (0, CHUNK_IDX - ROWS)))
    return _cbow_sc(idx.reshape(-1), table)
